# baseline (speedup 1.0000x reference)
import functools

import jax
import jax.numpy as jnp
from jax import lax
from jax.experimental import pallas as pl
from jax.experimental.pallas import tpu as pltpu
from jax.experimental.pallas import tpu_sc as plsc

_D = 64
_B = 4096
_H = 200

_NC = 2
_NS = 16
_NW = _NC * _NS
_TOTAL = _B * _H
_PER_W = _TOTAL // _NW
_CHUNK = 128
_NCHUNK = _PER_W // _CHUNK


def _embed_gather(weight, idx3):
  mesh = plsc.VectorSubcoreMesh(core_axis_name="c", subcore_axis_name="s")

  @functools.partial(
      pl.kernel,
      mesh=mesh,
      out_type=jax.ShapeDtypeStruct((_TOTAL, _D), jnp.float32),
      compiler_params=pltpu.CompilerParams(use_tc_tiling_on_sc=False),
      scratch_types=[
          pltpu.VMEM((_NCHUNK, _CHUNK), jnp.int32),
          pltpu.VMEM((_CHUNK, _D), jnp.float32),
          pltpu.VMEM((_CHUNK, _D), jnp.float32),
          pltpu.SemaphoreType.DMA,
          pltpu.SemaphoreType.DMA,
      ],
  )
  def k(table_hbm, idx_hbm, out_hbm, idx_v, buf0, buf1, sem0, sem1):
    wid = lax.axis_index("s") * _NC + lax.axis_index("c")
    base = wid * _PER_W
    pltpu.sync_copy(idx_hbm.at[wid], idx_v)

    def body(i, carry):
      a = 2 * i
      b = a + 1
      ca = pltpu.async_copy(table_hbm.at[idx_v.at[a]], buf0, sem0)
      cb = pltpu.async_copy(table_hbm.at[idx_v.at[b]], buf1, sem1)
      ca.wait()
      pltpu.sync_copy(buf0, out_hbm.at[pl.ds(base + a * _CHUNK, _CHUNK)])
      cb.wait()
      pltpu.sync_copy(buf1, out_hbm.at[pl.ds(base + b * _CHUNK, _CHUNK)])
      return carry

    lax.fori_loop(0, _NCHUNK // 2, body, 0)

  return k(weight, idx3)


def kernel(indices, weight):
  idx3 = indices.reshape(_NW, _NCHUNK, _CHUNK)
  out = _embed_gather(weight, idx3)
  return out.reshape(_B, _H, _D)


# speedup vs baseline: 1.0447x; 1.0447x over previous
import functools

import jax
import jax.numpy as jnp
from jax import lax
from jax.experimental import pallas as pl
from jax.experimental.pallas import tpu as pltpu
from jax.experimental.pallas import tpu_sc as plsc

_D = 64
_B = 4096
_H = 200

_NC = 2
_NS = 16
_NW = _NC * _NS
_TOTAL = _B * _H
_PER_W = _TOTAL // _NW
_CHUNK = 128
_NCHUNK = _PER_W // _CHUNK
_K = 5
_GROUP = _K * _CHUNK
_NG = _NCHUNK // _K
_NIT = _NG // 2


def _embed_gather(weight, idx3):
  mesh = plsc.VectorSubcoreMesh(core_axis_name="c", subcore_axis_name="s")

  @functools.partial(
      pl.kernel,
      mesh=mesh,
      out_type=jax.ShapeDtypeStruct((_TOTAL, _D), jnp.float32),
      compiler_params=pltpu.CompilerParams(use_tc_tiling_on_sc=False),
      scratch_types=[
          pltpu.VMEM((_NCHUNK, _CHUNK), jnp.int32),
          pltpu.VMEM((_GROUP, _D), jnp.float32),
          pltpu.VMEM((_GROUP, _D), jnp.float32),
          pltpu.SemaphoreType.DMA,
          pltpu.SemaphoreType.DMA,
          pltpu.SemaphoreType.DMA,
          pltpu.SemaphoreType.DMA,
      ],
  )
  def k(table_hbm, idx_hbm, out_hbm, idx_v, bufa, bufb, g0, g1, o0, o1):
    wid = lax.axis_index("s") * _NC + lax.axis_index("c")
    base = wid * _PER_W
    pltpu.sync_copy(idx_hbm.at[wid], idx_v)

    def fire_gather(group, buf, sem):
      for j in range(_K):
        pltpu.async_copy(
            table_hbm.at[idx_v.at[group * _K + j]],
            buf.at[pl.ds(j * _CHUNK, _CHUNK)], sem)

    def drain_gather(group, buf, sem):
      for j in range(_K):
        pltpu.make_async_copy(
            table_hbm.at[idx_v.at[group * _K + j]],
            buf.at[pl.ds(j * _CHUNK, _CHUNK)], sem).wait()

    def fire_out(group, buf, sem):
      pltpu.async_copy(
          buf, out_hbm.at[pl.ds(base + group * _GROUP, _GROUP)], sem)

    def drain_out(group, buf, sem):
      pltpu.make_async_copy(
          buf, out_hbm.at[pl.ds(base + group * _GROUP, _GROUP)], sem).wait()

    fire_gather(0, bufa, g0)

    def body(t, carry):
      a = 2 * t
      b = a + 1

      @pl.when(t > 0)
      def _():
        drain_out(b - 2, bufb, o1)

      fire_gather(b, bufb, g1)
      drain_gather(a, bufa, g0)
      fire_out(a, bufa, o0)
      drain_out(a, bufa, o0)

      @pl.when(t < _NIT - 1)
      def _():
        fire_gather(a + 2, bufa, g0)

      drain_gather(b, bufb, g1)
      fire_out(b, bufb, o1)
      return carry

    lax.fori_loop(0, _NIT, body, 0)
    drain_out(_NG - 1, bufb, o1)

  return k(weight, idx3)


def kernel(indices, weight):
  idx3 = indices.reshape(_NW, _NCHUNK, _CHUNK)
  out = _embed_gather(weight, idx3)
  return out.reshape(_B, _H, _D)
